# baseline (speedup 1.0000x reference)
import jax
import jax.numpy as jnp
from jax import lax
from jax.experimental import pallas as pl
from jax.experimental.pallas import tpu as pltpu
from jax.experimental.pallas import tpu_sc as plsc

N_NODES = 50000
D = 256
B = 4096
S = 10
N1 = B * S

NC = 2
NS = 16
NW = NC * NS

L = 16
KD = D // L
KP = D // 32

_MASK_HI = -65536


_ROWS1 = 1000


def _pack_pairs(x):
    a = x[:, :D // 2]
    b = x[:, D // 2:]

    def rnd(v):
        vi = lax.bitcast_convert_type(v, jnp.int32)
        return vi + 0x7FFF + (lax.shift_right_logical(vi, 16) & 1)

    return (rnd(b) & _MASK_HI) | lax.shift_right_logical(rnd(a), 16)


def _transform_body(x_ref, w_ref, p_ref):
    x = x_ref[...].astype(jnp.bfloat16)
    w = w_ref[...][0].astype(jnp.bfloat16)
    dn = (((1,), (1,)), ((), ()))
    p = lax.dot_general(x, w, dn, preferred_element_type=jnp.float32)
    p_ref[...] = _pack_pairs(p)


def _transform(features, w_stack):
    grid = N_NODES // _ROWS1
    return pl.pallas_call(
        _transform_body,
        grid=(2, grid),
        in_specs=[
            pl.BlockSpec((_ROWS1, D), lambda t, i: (i, 0)),
            pl.BlockSpec((1, D, D), lambda t, i: (t, 0, 0)),
        ],
        out_specs=pl.BlockSpec((_ROWS1, D // 2), lambda t, i: (t * grid + i, 0)),
        out_shape=jax.ShapeDtypeStruct((2 * N_NODES, D // 2), jnp.int32),
    )(features, w_stack)


_C1 = 8
_RPW1 = N1 // NW
_NCH1 = _RPW1 // _C1
_NB = 4
_B_PW = B // NW


def _unpack_lo(x):
    return lax.bitcast_convert_type(lax.shift_left(x, 16), jnp.float32)


def _unpack_hi(x):
    return lax.bitcast_convert_type(x & _MASK_HI, jnp.float32)


_S1 = S + 1


def _hop1_body(p_hbm, cidx_hbm, feat_hbm, nodes_hbm,
               h1_hbm, self2_hbm,
               cidx_v,
               nb0, nb1, nb2, nb3, ob0, ob1, ob2, ob3,
               s2idx, s2buf,
               g0, g1, g2, g3, o0, o1, o2, o3, s2sem):
    wid = lax.axis_index("s") * NC + lax.axis_index("c")
    base0 = wid * _RPW1
    nbufs = (nb0, nb1, nb2, nb3)
    obufs = (ob0, ob1, ob2, ob3)
    gsems = (g0, g1, g2, g3)
    osems = (o0, o1, o2, o3)

    sbase = wid * _B_PW
    pltpu.sync_copy(nodes_hbm.at[pl.ds(sbase, _B_PW)], s2idx)
    pltpu.async_copy(feat_hbm.at[s2idx], s2buf, s2sem)

    pltpu.sync_copy(cidx_hbm.at[pl.ds(base0 * _S1, _RPW1 * _S1)], cidx_v)

    def issue(c, j):
        pltpu.async_copy(p_hbm.at[cidx_v.at[pl.ds(c * (_C1 * _S1), _C1 * _S1)]],
                         nbufs[j], gsems[j])

    def wait_gather(j):
        pltpu.make_async_copy(p_hbm.at[cidx_v.at[pl.ds(0, _C1 * _S1)]],
                              nbufs[j], gsems[j]).wait()

    for j in range(_NB):
        issue(j, j)

    def step(i, _):
        for j in range(_NB):
            c = _NB * i + j
            wait_gather(j)

            @pl.when(c >= _NB)
            def _w():
                pltpu.make_async_copy(obufs[j], h1_hbm.at[pl.ds(base0, _C1)],
                                      osems[j]).wait()

            nb, ob = nbufs[j], obufs[j]

            def crow(r, _c):
                for k in range(KP):
                    xs = nb[r * _S1 + S, pl.ds(k * L, L)]
                    lo = _unpack_lo(xs)
                    hi = _unpack_hi(xs)
                    for s in range(S):
                        x = nb[r * _S1 + s, pl.ds(k * L, L)]
                        lo = lo + _unpack_lo(x)
                        hi = hi + _unpack_hi(x)
                    ob[r, pl.ds(k * L, L)] = jnp.maximum(lo, 0.0)
                    ob[r, pl.ds(D // 2 + k * L, L)] = jnp.maximum(hi, 0.0)
                return _c

            lax.fori_loop(0, _C1, crow, None)
            pltpu.async_copy(ob, h1_hbm.at[pl.ds(base0 + c * _C1, _C1)],
                             osems[j])

            @pl.when(c + _NB < _NCH1)
            def _i():
                issue(c + _NB, j)
        return _

    lax.fori_loop(0, _NCH1 // _NB, step, None)

    for j in range(_NB):
        pltpu.make_async_copy(obufs[j], h1_hbm.at[pl.ds(base0, _C1)],
                              osems[j]).wait()

    pltpu.make_async_copy(feat_hbm.at[s2idx], s2buf, s2sem).wait()
    pltpu.sync_copy(s2buf, self2_hbm.at[pl.ds(sbase, _B_PW)])


def _hop1(p, cidx_flat, features, nodes):
    mesh = plsc.VectorSubcoreMesh(core_axis_name="c", subcore_axis_name="s",
                                  num_cores=NC, num_subcores=NS)
    f = pl.kernel(
        _hop1_body,
        out_type=[
            jax.ShapeDtypeStruct((N1, D), jnp.float32),
            jax.ShapeDtypeStruct((B, D), jnp.float32),
        ],
        mesh=mesh,
        scratch_types=(
            [pltpu.VMEM((_RPW1 * _S1,), jnp.int32)]
            + [pltpu.VMEM((_C1 * _S1, D // 2), jnp.int32) for _ in range(_NB)]
            + [pltpu.VMEM((_C1, D), jnp.float32) for _ in range(_NB)]
            + [pltpu.VMEM((_B_PW,), jnp.int32),
               pltpu.VMEM((_B_PW, D), jnp.float32)]
            + [pltpu.SemaphoreType.DMA] * (2 * _NB + 1)
        ),
    )
    return f(p, cidx_flat, features, nodes)


_RPW2 = B // NW
_NCH2 = _RPW2 // _C1


def _hop2_body(h1_hbm, pidx_hbm, agg_hbm,
               pidx_v, nb0, nb1, nb2, nb3, ob0, ob1, ob2, ob3,
               g0, g1, g2, g3, o0, o1, o2, o3):
    wid = lax.axis_index("s") * NC + lax.axis_index("c")
    base0 = wid * _RPW2
    nbufs = (nb0, nb1, nb2, nb3)
    obufs = (ob0, ob1, ob2, ob3)
    gsems = (g0, g1, g2, g3)
    osems = (o0, o1, o2, o3)

    pltpu.sync_copy(pidx_hbm.at[pl.ds(base0 * S, _RPW2 * S)], pidx_v)

    def issue(c, j):
        pltpu.async_copy(h1_hbm.at[pidx_v.at[pl.ds(c * (_C1 * S), _C1 * S)]],
                         nbufs[j], gsems[j])

    def wait_gather(j):
        pltpu.make_async_copy(h1_hbm.at[pidx_v.at[pl.ds(0, _C1 * S)]],
                              nbufs[j], gsems[j]).wait()

    for j in range(_NB):
        issue(j, j)

    def step(i, _):
        for j in range(_NB):
            c = _NB * i + j
            wait_gather(j)

            @pl.when(c >= _NB)
            def _w():
                pltpu.make_async_copy(obufs[j], agg_hbm.at[pl.ds(base0, _C1)],
                                      osems[j]).wait()

            nb, ob = nbufs[j], obufs[j]

            def crow(r, _c):
                for k in range(KD):
                    acc = nb[r * S, pl.ds(k * L, L)]
                    for s in range(1, S):
                        acc = acc + nb[r * S + s, pl.ds(k * L, L)]
                    ob[r, pl.ds(k * L, L)] = acc
                return _c

            lax.fori_loop(0, _C1, crow, None)
            pltpu.async_copy(ob, agg_hbm.at[pl.ds(base0 + c * _C1, _C1)],
                             osems[j])

            @pl.when(c + _NB < _NCH2)
            def _i():
                issue(c + _NB, j)
        return _

    lax.fori_loop(0, _NCH2 // _NB, step, None)

    for j in range(_NB):
        pltpu.make_async_copy(obufs[j], agg_hbm.at[pl.ds(base0, _C1)],
                              osems[j]).wait()


def _hop2(h1p, npos_flat):
    mesh = plsc.VectorSubcoreMesh(core_axis_name="c", subcore_axis_name="s",
                                  num_cores=NC, num_subcores=NS)
    f = pl.kernel(
        _hop2_body,
        out_type=jax.ShapeDtypeStruct((B, D), jnp.float32),
        mesh=mesh,
        scratch_types=(
            [pltpu.VMEM((_RPW2 * S,), jnp.int32)]
            + [pltpu.VMEM((_C1 * S, D), jnp.float32) for _ in range(_NB)]
            + [pltpu.VMEM((_C1, D), jnp.float32) for _ in range(_NB)]
            + [pltpu.SemaphoreType.DMA] * (2 * _NB)
        ),
    )
    return f(h1p, npos_flat)


_ROWS4 = 1024


def _out_body(agg_ref, slf_ref, wn_ref, ws_ref, o_ref):
    dn = (((1,), (1,)), ((), ()))
    x = lax.dot_general(agg_ref[...], wn_ref[...], dn,
                        preferred_element_type=jnp.float32) * (1.0 / S)
    y = lax.dot_general(slf_ref[...], ws_ref[...], dn,
                        preferred_element_type=jnp.float32)
    o_ref[...] = jnp.maximum(x + y, 0.0)


def _out_proj(agg2, self2, w2n, w2s):
    grid = B // _ROWS4
    return pl.pallas_call(
        _out_body,
        grid=(grid,),
        in_specs=[
            pl.BlockSpec((_ROWS4, D), lambda i: (i, 0)),
            pl.BlockSpec((_ROWS4, D), lambda i: (i, 0)),
            pl.BlockSpec((D, D), lambda i: (0, 0)),
            pl.BlockSpec((D, D), lambda i: (0, 0)),
        ],
        out_specs=pl.BlockSpec((_ROWS4, D), lambda i: (i, 0)),
        out_shape=jax.ShapeDtypeStruct((B, D), jnp.float32),
    )(agg2, self2, w2n, w2s)


def kernel(features, nodes, nodes_l1, neigh_l1, neigh_pos, W1_nei, W1_self,
           W2_nei, W2_self):
    nodes = nodes.astype(jnp.int32)
    npos_flat = neigh_pos.reshape(-1)
    cidx_flat = jnp.concatenate(
        [neigh_l1, nodes_l1[:, None] + N_NODES], axis=1).reshape(-1)
    w_stack = jnp.stack([W1_nei * (1.0 / S), W1_self])

    p = _transform(features, w_stack)
    h1, self2 = _hop1(p, cidx_flat, features, nodes)
    agg2 = _hop2(h1, npos_flat)
    return _out_proj(agg2, self2, W2_nei, W2_self)


# speedup vs baseline: 1.0963x; 1.0963x over previous
import jax
import jax.numpy as jnp
from jax import lax
from jax.experimental import pallas as pl
from jax.experimental.pallas import tpu as pltpu
from jax.experimental.pallas import tpu_sc as plsc

N_NODES = 50000
D = 256
B = 4096
S = 10
N1 = B * S

NC = 2
NS = 16
NW = NC * NS

L = 16
KD = D // L
KP = D // 32

_MASK_HI = -65536


_ROWS1 = 1000


def _pack_pairs(x):
    a = x[:, :D // 2]
    b = x[:, D // 2:]

    def rnd(v):
        vi = lax.bitcast_convert_type(v, jnp.int32)
        return vi + 0x7FFF + (lax.shift_right_logical(vi, 16) & 1)

    return (rnd(b) & _MASK_HI) | lax.shift_right_logical(rnd(a), 16)


def _transform_body(x_ref, wn_ref, ws_ref, p_ref):
    x = x_ref[...].astype(jnp.bfloat16)
    dn = (((1,), (1,)), ((), ()))
    pn = lax.dot_general(x, wn_ref[...].astype(jnp.bfloat16), dn,
                         preferred_element_type=jnp.float32)
    ps = lax.dot_general(x, ws_ref[...].astype(jnp.bfloat16), dn,
                         preferred_element_type=jnp.float32)
    p_ref[0] = _pack_pairs(pn)
    p_ref[1] = _pack_pairs(ps)


def _transform(features, wn, ws):
    grid = N_NODES // _ROWS1
    return pl.pallas_call(
        _transform_body,
        grid=(grid,),
        in_specs=[
            pl.BlockSpec((_ROWS1, D), lambda i: (i, 0)),
            pl.BlockSpec((D, D), lambda i: (0, 0)),
            pl.BlockSpec((D, D), lambda i: (0, 0)),
        ],
        out_specs=pl.BlockSpec((2, _ROWS1, D // 2), lambda i: (0, i, 0)),
        out_shape=jax.ShapeDtypeStruct((2, N_NODES, D // 2), jnp.int32),
    )(features, wn, ws)


_C1 = 8
_RPW1 = N1 // NW
_NCH1 = _RPW1 // _C1
_NB = 4
_B_PW = B // NW


def _unpack_lo(x):
    return lax.bitcast_convert_type(lax.shift_left(x, 16), jnp.float32)


def _unpack_hi(x):
    return lax.bitcast_convert_type(x & _MASK_HI, jnp.float32)


_S1 = S + 1


def _hop1_body(p_hbm, cidx_hbm, feat_hbm, nodes_hbm,
               h1_hbm, self2_hbm,
               cidx_v,
               nb0, nb1, nb2, nb3, ob0, ob1, ob2, ob3,
               s2idx, s2buf,
               g0, g1, g2, g3, o0, o1, o2, o3, s2sem):
    wid = lax.axis_index("s") * NC + lax.axis_index("c")
    base0 = wid * _RPW1
    nbufs = (nb0, nb1, nb2, nb3)
    obufs = (ob0, ob1, ob2, ob3)
    gsems = (g0, g1, g2, g3)
    osems = (o0, o1, o2, o3)

    sbase = wid * _B_PW
    pltpu.sync_copy(nodes_hbm.at[pl.ds(sbase, _B_PW)], s2idx)
    pltpu.async_copy(feat_hbm.at[s2idx], s2buf, s2sem)

    pltpu.sync_copy(cidx_hbm.at[pl.ds(base0 * _S1, _RPW1 * _S1)], cidx_v)

    def issue(c, j):
        pltpu.async_copy(p_hbm.at[cidx_v.at[pl.ds(c * (_C1 * _S1), _C1 * _S1)]],
                         nbufs[j], gsems[j])

    def wait_gather(j):
        pltpu.make_async_copy(p_hbm.at[cidx_v.at[pl.ds(0, _C1 * _S1)]],
                              nbufs[j], gsems[j]).wait()

    for j in range(_NB):
        issue(j, j)

    def step(i, _):
        for j in range(_NB):
            c = _NB * i + j
            wait_gather(j)

            @pl.when(c >= _NB)
            def _w():
                pltpu.make_async_copy(obufs[j], h1_hbm.at[pl.ds(base0, _C1)],
                                      osems[j]).wait()

            nb, ob = nbufs[j], obufs[j]

            def crow(r, _c):
                for k in range(KP):
                    xs = nb[r * _S1 + S, pl.ds(k * L, L)]
                    lo = _unpack_lo(xs)
                    hi = _unpack_hi(xs)
                    for s in range(S):
                        x = nb[r * _S1 + s, pl.ds(k * L, L)]
                        lo = lo + _unpack_lo(x)
                        hi = hi + _unpack_hi(x)
                    ob[r, pl.ds(k * L, L)] = jnp.maximum(lo, 0.0)
                    ob[r, pl.ds(D // 2 + k * L, L)] = jnp.maximum(hi, 0.0)
                return _c

            lax.fori_loop(0, _C1, crow, None)
            pltpu.async_copy(ob, h1_hbm.at[pl.ds(base0 + c * _C1, _C1)],
                             osems[j])

            @pl.when(c + _NB < _NCH1)
            def _i():
                issue(c + _NB, j)
        return _

    lax.fori_loop(0, _NCH1 // _NB, step, None)

    for j in range(_NB):
        pltpu.make_async_copy(obufs[j], h1_hbm.at[pl.ds(base0, _C1)],
                              osems[j]).wait()

    pltpu.make_async_copy(feat_hbm.at[s2idx], s2buf, s2sem).wait()
    pltpu.sync_copy(s2buf, self2_hbm.at[pl.ds(sbase, _B_PW)])


def _hop1(p, cidx_flat, features, nodes):
    mesh = plsc.VectorSubcoreMesh(core_axis_name="c", subcore_axis_name="s",
                                  num_cores=NC, num_subcores=NS)
    f = pl.kernel(
        _hop1_body,
        out_type=[
            jax.ShapeDtypeStruct((N1, D), jnp.float32),
            jax.ShapeDtypeStruct((B, D), jnp.float32),
        ],
        mesh=mesh,
        scratch_types=(
            [pltpu.VMEM((_RPW1 * _S1,), jnp.int32)]
            + [pltpu.VMEM((_C1 * _S1, D // 2), jnp.int32) for _ in range(_NB)]
            + [pltpu.VMEM((_C1, D), jnp.float32) for _ in range(_NB)]
            + [pltpu.VMEM((_B_PW,), jnp.int32),
               pltpu.VMEM((_B_PW, D), jnp.float32)]
            + [pltpu.SemaphoreType.DMA] * (2 * _NB + 1)
        ),
    )
    return f(p, cidx_flat, features, nodes)


_RPW2 = B // NW
_NCH2 = _RPW2 // _C1


def _hop2_body(h1_hbm, pidx_hbm, agg_hbm,
               pidx_v, nb0, nb1, nb2, nb3, ob0, ob1, ob2, ob3,
               g0, g1, g2, g3, o0, o1, o2, o3):
    wid = lax.axis_index("s") * NC + lax.axis_index("c")
    base0 = wid * _RPW2
    nbufs = (nb0, nb1, nb2, nb3)
    obufs = (ob0, ob1, ob2, ob3)
    gsems = (g0, g1, g2, g3)
    osems = (o0, o1, o2, o3)

    pltpu.sync_copy(pidx_hbm.at[pl.ds(base0 * S, _RPW2 * S)], pidx_v)

    def issue(c, j):
        pltpu.async_copy(h1_hbm.at[pidx_v.at[pl.ds(c * (_C1 * S), _C1 * S)]],
                         nbufs[j], gsems[j])

    def wait_gather(j):
        pltpu.make_async_copy(h1_hbm.at[pidx_v.at[pl.ds(0, _C1 * S)]],
                              nbufs[j], gsems[j]).wait()

    for j in range(_NB):
        issue(j, j)

    def step(i, _):
        for j in range(_NB):
            c = _NB * i + j
            wait_gather(j)

            @pl.when(c >= _NB)
            def _w():
                pltpu.make_async_copy(obufs[j], agg_hbm.at[pl.ds(base0, _C1)],
                                      osems[j]).wait()

            nb, ob = nbufs[j], obufs[j]

            def crow(r, _c):
                for k in range(KD):
                    acc = nb[r * S, pl.ds(k * L, L)]
                    for s in range(1, S):
                        acc = acc + nb[r * S + s, pl.ds(k * L, L)]
                    ob[r, pl.ds(k * L, L)] = acc
                return _c

            lax.fori_loop(0, _C1, crow, None)
            pltpu.async_copy(ob, agg_hbm.at[pl.ds(base0 + c * _C1, _C1)],
                             osems[j])

            @pl.when(c + _NB < _NCH2)
            def _i():
                issue(c + _NB, j)
        return _

    lax.fori_loop(0, _NCH2 // _NB, step, None)

    for j in range(_NB):
        pltpu.make_async_copy(obufs[j], agg_hbm.at[pl.ds(base0, _C1)],
                              osems[j]).wait()


def _hop2(h1p, npos_flat):
    mesh = plsc.VectorSubcoreMesh(core_axis_name="c", subcore_axis_name="s",
                                  num_cores=NC, num_subcores=NS)
    f = pl.kernel(
        _hop2_body,
        out_type=jax.ShapeDtypeStruct((B, D), jnp.float32),
        mesh=mesh,
        scratch_types=(
            [pltpu.VMEM((_RPW2 * S,), jnp.int32)]
            + [pltpu.VMEM((_C1 * S, D), jnp.float32) for _ in range(_NB)]
            + [pltpu.VMEM((_C1, D), jnp.float32) for _ in range(_NB)]
            + [pltpu.SemaphoreType.DMA] * (2 * _NB)
        ),
    )
    return f(h1p, npos_flat)


_ROWS4 = 1024


def _out_body(agg_ref, slf_ref, wn_ref, ws_ref, o_ref):
    dn = (((1,), (1,)), ((), ()))
    x = lax.dot_general(agg_ref[...], wn_ref[...], dn,
                        preferred_element_type=jnp.float32) * (1.0 / S)
    y = lax.dot_general(slf_ref[...], ws_ref[...], dn,
                        preferred_element_type=jnp.float32)
    o_ref[...] = jnp.maximum(x + y, 0.0)


def _out_proj(agg2, self2, w2n, w2s):
    grid = B // _ROWS4
    return pl.pallas_call(
        _out_body,
        grid=(grid,),
        in_specs=[
            pl.BlockSpec((_ROWS4, D), lambda i: (i, 0)),
            pl.BlockSpec((_ROWS4, D), lambda i: (i, 0)),
            pl.BlockSpec((D, D), lambda i: (0, 0)),
            pl.BlockSpec((D, D), lambda i: (0, 0)),
        ],
        out_specs=pl.BlockSpec((_ROWS4, D), lambda i: (i, 0)),
        out_shape=jax.ShapeDtypeStruct((B, D), jnp.float32),
    )(agg2, self2, w2n, w2s)


def kernel(features, nodes, nodes_l1, neigh_l1, neigh_pos, W1_nei, W1_self,
           W2_nei, W2_self):
    nodes = nodes.astype(jnp.int32)
    npos_flat = neigh_pos.reshape(-1)
    cidx_flat = jnp.concatenate(
        [neigh_l1, nodes_l1[:, None] + N_NODES], axis=1).reshape(-1)
    p = _transform(features, W1_nei * (1.0 / S), W1_self)
    p = p.reshape(2 * N_NODES, D // 2)
    h1, self2 = _hop1(p, cidx_flat, features, nodes)
    agg2 = _hop2(h1, npos_flat)
    return _out_proj(agg2, self2, W2_nei, W2_self)


# speedup vs baseline: 1.1430x; 1.0426x over previous
import jax
import jax.numpy as jnp
from jax import lax
from jax.experimental import pallas as pl
from jax.experimental.pallas import tpu as pltpu
from jax.experimental.pallas import tpu_sc as plsc

N_NODES = 50000
D = 256
B = 4096
S = 10
N1 = B * S

NC = 2
NS = 16
NW = NC * NS

L = 16
KD = D // L
KP = D // 32

_MASK_HI = -65536


_ROWS1 = 2000


def _pack_pairs(x):
    a = x[:, :D // 2]
    b = x[:, D // 2:]

    def rnd(v):
        vi = lax.bitcast_convert_type(v, jnp.int32)
        return vi + 0x7FFF + (lax.shift_right_logical(vi, 16) & 1)

    return (rnd(b) & _MASK_HI) | lax.shift_right_logical(rnd(a), 16)


def _transform_body(x_ref, wn_ref, ws_ref, p_ref):
    x = x_ref[...].astype(jnp.bfloat16)
    dn = (((1,), (1,)), ((), ()))
    pn = lax.dot_general(x, wn_ref[...].astype(jnp.bfloat16), dn,
                         preferred_element_type=jnp.float32)
    ps = lax.dot_general(x, ws_ref[...].astype(jnp.bfloat16), dn,
                         preferred_element_type=jnp.float32)
    p_ref[0] = _pack_pairs(pn)
    p_ref[1] = _pack_pairs(ps)


def _transform(features, wn, ws):
    grid = N_NODES // _ROWS1
    return pl.pallas_call(
        _transform_body,
        grid=(grid,),
        in_specs=[
            pl.BlockSpec((_ROWS1, D), lambda i: (i, 0)),
            pl.BlockSpec((D, D), lambda i: (0, 0)),
            pl.BlockSpec((D, D), lambda i: (0, 0)),
        ],
        out_specs=pl.BlockSpec((2, _ROWS1, D // 2), lambda i: (0, i, 0)),
        out_shape=jax.ShapeDtypeStruct((2, N_NODES, D // 2), jnp.int32),
    )(features, wn, ws)


_C1 = 8
_RPW1 = N1 // NW
_NCH1 = _RPW1 // _C1
_NB = 4
_B_PW = B // NW


def _unpack_lo(x):
    return lax.bitcast_convert_type(lax.shift_left(x, 16), jnp.float32)


def _unpack_hi(x):
    return lax.bitcast_convert_type(x & _MASK_HI, jnp.float32)


_S1 = S + 1


def _hop1_body(p_hbm, cidx_hbm, feat_hbm, nodes_hbm,
               h1_hbm, self2_hbm,
               cidx_v,
               nb0, nb1, nb2, nb3, ob0, ob1, ob2, ob3,
               s2idx, s2buf,
               g0, g1, g2, g3, o0, o1, o2, o3, s2sem):
    wid = lax.axis_index("s") * NC + lax.axis_index("c")
    base0 = wid * _RPW1
    nbufs = (nb0, nb1, nb2, nb3)
    obufs = (ob0, ob1, ob2, ob3)
    gsems = (g0, g1, g2, g3)
    osems = (o0, o1, o2, o3)

    sbase = wid * _B_PW
    pltpu.sync_copy(nodes_hbm.at[pl.ds(sbase, _B_PW)], s2idx)
    pltpu.async_copy(feat_hbm.at[s2idx], s2buf, s2sem)

    pltpu.sync_copy(cidx_hbm.at[pl.ds(base0 * _S1, _RPW1 * _S1)], cidx_v)

    def issue(c, j):
        pltpu.async_copy(p_hbm.at[cidx_v.at[pl.ds(c * (_C1 * _S1), _C1 * _S1)]],
                         nbufs[j], gsems[j])

    def wait_gather(j):
        pltpu.make_async_copy(p_hbm.at[cidx_v.at[pl.ds(0, _C1 * _S1)]],
                              nbufs[j], gsems[j]).wait()

    for j in range(_NB):
        issue(j, j)

    def step(i, _):
        for j in range(_NB):
            c = _NB * i + j
            wait_gather(j)

            @pl.when(c >= _NB)
            def _w():
                pltpu.make_async_copy(obufs[j], h1_hbm.at[pl.ds(base0, _C1)],
                                      osems[j]).wait()

            nb, ob = nbufs[j], obufs[j]

            def crow(r, _c):
                for k in range(KP):
                    xs = nb[r * _S1 + S, pl.ds(k * L, L)]
                    lo = _unpack_lo(xs)
                    hi = _unpack_hi(xs)
                    for s in range(S):
                        x = nb[r * _S1 + s, pl.ds(k * L, L)]
                        lo = lo + _unpack_lo(x)
                        hi = hi + _unpack_hi(x)
                    ob[r, pl.ds(k * L, L)] = jnp.maximum(lo, 0.0)
                    ob[r, pl.ds(D // 2 + k * L, L)] = jnp.maximum(hi, 0.0)
                return _c

            lax.fori_loop(0, _C1, crow, None)
            pltpu.async_copy(ob, h1_hbm.at[pl.ds(base0 + c * _C1, _C1)],
                             osems[j])

            @pl.when(c + _NB < _NCH1)
            def _i():
                issue(c + _NB, j)
        return _

    lax.fori_loop(0, _NCH1 // _NB, step, None)

    for j in range(_NB):
        pltpu.make_async_copy(obufs[j], h1_hbm.at[pl.ds(base0, _C1)],
                              osems[j]).wait()

    pltpu.make_async_copy(feat_hbm.at[s2idx], s2buf, s2sem).wait()
    pltpu.sync_copy(s2buf, self2_hbm.at[pl.ds(sbase, _B_PW)])


def _hop1(p, cidx_flat, features, nodes):
    mesh = plsc.VectorSubcoreMesh(core_axis_name="c", subcore_axis_name="s",
                                  num_cores=NC, num_subcores=NS)
    f = pl.kernel(
        _hop1_body,
        out_type=[
            jax.ShapeDtypeStruct((N1, D), jnp.float32),
            jax.ShapeDtypeStruct((B, D), jnp.float32),
        ],
        mesh=mesh,
        scratch_types=(
            [pltpu.VMEM((_RPW1 * _S1,), jnp.int32)]
            + [pltpu.VMEM((_C1 * _S1, D // 2), jnp.int32) for _ in range(_NB)]
            + [pltpu.VMEM((_C1, D), jnp.float32) for _ in range(_NB)]
            + [pltpu.VMEM((_B_PW,), jnp.int32),
               pltpu.VMEM((_B_PW, D), jnp.float32)]
            + [pltpu.SemaphoreType.DMA] * (2 * _NB + 1)
        ),
    )
    return f(p, cidx_flat, features, nodes)


_RPW2 = B // NW
_NCH2 = _RPW2 // _C1


def _hop2_body(h1_hbm, pidx_hbm, agg_hbm,
               pidx_v, nb0, nb1, nb2, nb3, ob0, ob1, ob2, ob3,
               g0, g1, g2, g3, o0, o1, o2, o3):
    wid = lax.axis_index("s") * NC + lax.axis_index("c")
    base0 = wid * _RPW2
    nbufs = (nb0, nb1, nb2, nb3)
    obufs = (ob0, ob1, ob2, ob3)
    gsems = (g0, g1, g2, g3)
    osems = (o0, o1, o2, o3)

    pltpu.sync_copy(pidx_hbm.at[pl.ds(base0 * S, _RPW2 * S)], pidx_v)

    def issue(c, j):
        pltpu.async_copy(h1_hbm.at[pidx_v.at[pl.ds(c * (_C1 * S), _C1 * S)]],
                         nbufs[j], gsems[j])

    def wait_gather(j):
        pltpu.make_async_copy(h1_hbm.at[pidx_v.at[pl.ds(0, _C1 * S)]],
                              nbufs[j], gsems[j]).wait()

    for j in range(_NB):
        issue(j, j)

    def step(i, _):
        for j in range(_NB):
            c = _NB * i + j
            wait_gather(j)

            @pl.when(c >= _NB)
            def _w():
                pltpu.make_async_copy(obufs[j], agg_hbm.at[pl.ds(base0, _C1)],
                                      osems[j]).wait()

            nb, ob = nbufs[j], obufs[j]

            def crow(r, _c):
                for k in range(KD):
                    acc = nb[r * S, pl.ds(k * L, L)]
                    for s in range(1, S):
                        acc = acc + nb[r * S + s, pl.ds(k * L, L)]
                    ob[r, pl.ds(k * L, L)] = acc
                return _c

            lax.fori_loop(0, _C1, crow, None)
            pltpu.async_copy(ob, agg_hbm.at[pl.ds(base0 + c * _C1, _C1)],
                             osems[j])

            @pl.when(c + _NB < _NCH2)
            def _i():
                issue(c + _NB, j)
        return _

    lax.fori_loop(0, _NCH2 // _NB, step, None)

    for j in range(_NB):
        pltpu.make_async_copy(obufs[j], agg_hbm.at[pl.ds(base0, _C1)],
                              osems[j]).wait()


def _hop2(h1p, npos_flat):
    mesh = plsc.VectorSubcoreMesh(core_axis_name="c", subcore_axis_name="s",
                                  num_cores=NC, num_subcores=NS)
    f = pl.kernel(
        _hop2_body,
        out_type=jax.ShapeDtypeStruct((B, D), jnp.float32),
        mesh=mesh,
        scratch_types=(
            [pltpu.VMEM((_RPW2 * S,), jnp.int32)]
            + [pltpu.VMEM((_C1 * S, D), jnp.float32) for _ in range(_NB)]
            + [pltpu.VMEM((_C1, D), jnp.float32) for _ in range(_NB)]
            + [pltpu.SemaphoreType.DMA] * (2 * _NB)
        ),
    )
    return f(h1p, npos_flat)


_ROWS4 = 1024


def _out_body(agg_ref, slf_ref, wn_ref, ws_ref, o_ref):
    dn = (((1,), (1,)), ((), ()))
    x = lax.dot_general(agg_ref[...], wn_ref[...], dn,
                        preferred_element_type=jnp.float32) * (1.0 / S)
    y = lax.dot_general(slf_ref[...], ws_ref[...], dn,
                        preferred_element_type=jnp.float32)
    o_ref[...] = jnp.maximum(x + y, 0.0)


def _out_proj(agg2, self2, w2n, w2s):
    grid = B // _ROWS4
    return pl.pallas_call(
        _out_body,
        grid=(grid,),
        in_specs=[
            pl.BlockSpec((_ROWS4, D), lambda i: (i, 0)),
            pl.BlockSpec((_ROWS4, D), lambda i: (i, 0)),
            pl.BlockSpec((D, D), lambda i: (0, 0)),
            pl.BlockSpec((D, D), lambda i: (0, 0)),
        ],
        out_specs=pl.BlockSpec((_ROWS4, D), lambda i: (i, 0)),
        out_shape=jax.ShapeDtypeStruct((B, D), jnp.float32),
    )(agg2, self2, w2n, w2s)


def kernel(features, nodes, nodes_l1, neigh_l1, neigh_pos, W1_nei, W1_self,
           W2_nei, W2_self):
    nodes = nodes.astype(jnp.int32)
    npos_flat = neigh_pos.reshape(-1)
    cidx_flat = jnp.concatenate(
        [neigh_l1, nodes_l1[:, None] + N_NODES], axis=1).reshape(-1)
    p = _transform(features, W1_nei * (1.0 / S), W1_self)
    p = p.reshape(2 * N_NODES, D // 2)
    h1, self2 = _hop1(p, cidx_flat, features, nodes)
    agg2 = _hop2(h1, npos_flat)
    return _out_proj(agg2, self2, W2_nei, W2_self)


# speedup vs baseline: 1.1437x; 1.0006x over previous
import jax
import jax.numpy as jnp
from jax import lax
from jax.experimental import pallas as pl
from jax.experimental.pallas import tpu as pltpu
from jax.experimental.pallas import tpu_sc as plsc

N_NODES = 50000
D = 256
B = 4096
S = 10
N1 = B * S

NC = 2
NS = 16
NW = NC * NS

L = 16
KD = D // L
KP = D // 32

_MASK_HI = -65536


_ROWS1 = 2000


def _pack_pairs(x):
    a = x[:, :D // 2]
    b = x[:, D // 2:]

    def rnd(v):
        vi = lax.bitcast_convert_type(v, jnp.int32)
        return vi + 0x7FFF + (lax.shift_right_logical(vi, 16) & 1)

    return (rnd(b) & _MASK_HI) | lax.shift_right_logical(rnd(a), 16)


def _transform_body(x_ref, wn_ref, ws_ref, p_ref):
    x = x_ref[...].astype(jnp.bfloat16)
    dn = (((1,), (1,)), ((), ()))
    pn = lax.dot_general(x, wn_ref[...].astype(jnp.bfloat16), dn,
                         preferred_element_type=jnp.float32)
    ps = lax.dot_general(x, ws_ref[...].astype(jnp.bfloat16), dn,
                         preferred_element_type=jnp.float32)
    p_ref[0] = _pack_pairs(pn)
    p_ref[1] = _pack_pairs(ps)


def _transform(features, wn, ws):
    grid = N_NODES // _ROWS1
    return pl.pallas_call(
        _transform_body,
        grid=(grid,),
        in_specs=[
            pl.BlockSpec((_ROWS1, D), lambda i: (i, 0)),
            pl.BlockSpec((D, D), lambda i: (0, 0)),
            pl.BlockSpec((D, D), lambda i: (0, 0)),
        ],
        out_specs=pl.BlockSpec((2, _ROWS1, D // 2), lambda i: (0, i, 0)),
        out_shape=jax.ShapeDtypeStruct((2, N_NODES, D // 2), jnp.int32),
    )(features, wn, ws)


_C1 = 8
_RPW1 = N1 // NW
_NCH1 = _RPW1 // _C1
_NB = 4
_B_PW = B // NW


def _unpack_lo(x):
    return lax.bitcast_convert_type(lax.shift_left(x, 16), jnp.float32)


def _unpack_hi(x):
    return lax.bitcast_convert_type(x & _MASK_HI, jnp.float32)


_S1 = S + 1


def _hop1_body(p_hbm, cidx_hbm, feat_hbm, nodes_hbm,
               h1_hbm, self2_hbm,
               cidx_v,
               nb0, nb1, nb2, nb3, ob0, ob1, ob2, ob3,
               s2idx, s2buf,
               g0, g1, g2, g3, o0, o1, o2, o3, s2sem):
    wid = lax.axis_index("s") * NC + lax.axis_index("c")
    base0 = wid * _RPW1
    nbufs = (nb0, nb1, nb2, nb3)
    obufs = (ob0, ob1, ob2, ob3)
    gsems = (g0, g1, g2, g3)
    osems = (o0, o1, o2, o3)

    sbase = wid * _B_PW
    pltpu.sync_copy(nodes_hbm.at[pl.ds(sbase, _B_PW)], s2idx)
    pltpu.async_copy(feat_hbm.at[s2idx], s2buf, s2sem)

    pltpu.sync_copy(cidx_hbm.at[pl.ds(base0 * _S1, _RPW1 * _S1)], cidx_v)

    def issue(c, j):
        pltpu.async_copy(p_hbm.at[cidx_v.at[pl.ds(c * (_C1 * _S1), _C1 * _S1)]],
                         nbufs[j], gsems[j])

    def wait_gather(j):
        pltpu.make_async_copy(p_hbm.at[cidx_v.at[pl.ds(0, _C1 * _S1)]],
                              nbufs[j], gsems[j]).wait()

    for j in range(_NB):
        issue(j, j)

    def step(i, _):
        for j in range(_NB):
            c = _NB * i + j
            wait_gather(j)

            @pl.when(c >= _NB)
            def _w():
                pltpu.make_async_copy(obufs[j], h1_hbm.at[pl.ds(base0, _C1)],
                                      osems[j]).wait()

            nb, ob = nbufs[j], obufs[j]

            def crow(r, _c):
                for k in range(KP):
                    xs = nb[r * _S1 + S, pl.ds(k * L, L)]
                    lo = _unpack_lo(xs)
                    hi = _unpack_hi(xs)
                    for s in range(S):
                        x = nb[r * _S1 + s, pl.ds(k * L, L)]
                        lo = lo + _unpack_lo(x)
                        hi = hi + _unpack_hi(x)
                    ob[r, pl.ds(k * L, L)] = jnp.maximum(lo, 0.0)
                    ob[r, pl.ds(D // 2 + k * L, L)] = jnp.maximum(hi, 0.0)
                return _c

            lax.fori_loop(0, _C1, crow, None)
            pltpu.async_copy(ob, h1_hbm.at[pl.ds(base0 + c * _C1, _C1)],
                             osems[j])

            @pl.when(c + _NB < _NCH1)
            def _i():
                issue(c + _NB, j)
        return _

    lax.fori_loop(0, _NCH1 // _NB, step, None)

    for j in range(_NB):
        pltpu.make_async_copy(obufs[j], h1_hbm.at[pl.ds(base0, _C1)],
                              osems[j]).wait()

    pltpu.make_async_copy(feat_hbm.at[s2idx], s2buf, s2sem).wait()
    pltpu.sync_copy(s2buf, self2_hbm.at[pl.ds(sbase, _B_PW)])


def _hop1(p, cidx_flat, features, nodes):
    mesh = plsc.VectorSubcoreMesh(core_axis_name="c", subcore_axis_name="s",
                                  num_cores=NC, num_subcores=NS)
    f = pl.kernel(
        _hop1_body,
        out_type=[
            jax.ShapeDtypeStruct((N1, D), jnp.float32),
            jax.ShapeDtypeStruct((B, D), jnp.float32),
        ],
        mesh=mesh,
        scratch_types=(
            [pltpu.VMEM((_RPW1 * _S1,), jnp.int32)]
            + [pltpu.VMEM((_C1 * _S1, D // 2), jnp.int32) for _ in range(_NB)]
            + [pltpu.VMEM((_C1, D), jnp.float32) for _ in range(_NB)]
            + [pltpu.VMEM((_B_PW,), jnp.int32),
               pltpu.VMEM((_B_PW, D), jnp.float32)]
            + [pltpu.SemaphoreType.DMA] * (2 * _NB + 1)
        ),
    )
    return f(p, cidx_flat, features, nodes)


_RPW2 = B // NW
_NCH2 = _RPW2 // _C1


def _hop2_body(h1_hbm, pidx_hbm, agg_hbm,
               pidx_v, nb0, nb1, nb2, nb3, ob0, ob1, ob2, ob3,
               g0, g1, g2, g3, o0, o1, o2, o3):
    wid = lax.axis_index("s") * NC + lax.axis_index("c")
    base0 = wid * _RPW2
    nbufs = (nb0, nb1, nb2, nb3)
    obufs = (ob0, ob1, ob2, ob3)
    gsems = (g0, g1, g2, g3)
    osems = (o0, o1, o2, o3)

    pltpu.sync_copy(pidx_hbm.at[pl.ds(base0 * S, _RPW2 * S)], pidx_v)

    def issue(c, j):
        pltpu.async_copy(h1_hbm.at[pidx_v.at[pl.ds(c * (_C1 * S), _C1 * S)]],
                         nbufs[j], gsems[j])

    def wait_gather(j):
        pltpu.make_async_copy(h1_hbm.at[pidx_v.at[pl.ds(0, _C1 * S)]],
                              nbufs[j], gsems[j]).wait()

    for j in range(_NB):
        issue(j, j)

    def step(i, _):
        for j in range(_NB):
            c = _NB * i + j
            wait_gather(j)

            @pl.when(c >= _NB)
            def _w():
                pltpu.make_async_copy(obufs[j], agg_hbm.at[pl.ds(base0, _C1)],
                                      osems[j]).wait()

            nb, ob = nbufs[j], obufs[j]

            def crow(r, _c):
                for k in range(KD):
                    acc = nb[r * S, pl.ds(k * L, L)]
                    for s in range(1, S):
                        acc = acc + nb[r * S + s, pl.ds(k * L, L)]
                    ob[r, pl.ds(k * L, L)] = acc
                return _c

            lax.fori_loop(0, _C1, crow, None)
            pltpu.async_copy(ob, agg_hbm.at[pl.ds(base0 + c * _C1, _C1)],
                             osems[j])

            @pl.when(c + _NB < _NCH2)
            def _i():
                issue(c + _NB, j)
        return _

    lax.fori_loop(0, _NCH2 // _NB, step, None)

    for j in range(_NB):
        pltpu.make_async_copy(obufs[j], agg_hbm.at[pl.ds(base0, _C1)],
                              osems[j]).wait()


def _hop2(h1p, npos_flat):
    mesh = plsc.VectorSubcoreMesh(core_axis_name="c", subcore_axis_name="s",
                                  num_cores=NC, num_subcores=NS)
    f = pl.kernel(
        _hop2_body,
        out_type=jax.ShapeDtypeStruct((B, D), jnp.float32),
        mesh=mesh,
        scratch_types=(
            [pltpu.VMEM((_RPW2 * S,), jnp.int32)]
            + [pltpu.VMEM((_C1 * S, D), jnp.float32) for _ in range(_NB)]
            + [pltpu.VMEM((_C1, D), jnp.float32) for _ in range(_NB)]
            + [pltpu.SemaphoreType.DMA] * (2 * _NB)
        ),
    )
    return f(h1p, npos_flat)


_ROWS4 = 2048


def _out_body(agg_ref, slf_ref, wn_ref, ws_ref, o_ref):
    dn = (((1,), (1,)), ((), ()))
    x = lax.dot_general(agg_ref[...], wn_ref[...], dn,
                        preferred_element_type=jnp.float32) * (1.0 / S)
    y = lax.dot_general(slf_ref[...], ws_ref[...], dn,
                        preferred_element_type=jnp.float32)
    o_ref[...] = jnp.maximum(x + y, 0.0)


def _out_proj(agg2, self2, w2n, w2s):
    grid = B // _ROWS4
    return pl.pallas_call(
        _out_body,
        grid=(grid,),
        in_specs=[
            pl.BlockSpec((_ROWS4, D), lambda i: (i, 0)),
            pl.BlockSpec((_ROWS4, D), lambda i: (i, 0)),
            pl.BlockSpec((D, D), lambda i: (0, 0)),
            pl.BlockSpec((D, D), lambda i: (0, 0)),
        ],
        out_specs=pl.BlockSpec((_ROWS4, D), lambda i: (i, 0)),
        out_shape=jax.ShapeDtypeStruct((B, D), jnp.float32),
    )(agg2, self2, w2n, w2s)


def kernel(features, nodes, nodes_l1, neigh_l1, neigh_pos, W1_nei, W1_self,
           W2_nei, W2_self):
    nodes = nodes.astype(jnp.int32)
    npos_flat = neigh_pos.reshape(-1)
    cidx_flat = jnp.concatenate(
        [neigh_l1, nodes_l1[:, None] + N_NODES], axis=1).reshape(-1)
    p = _transform(features, W1_nei * (1.0 / S), W1_self)
    p = p.reshape(2 * N_NODES, D // 2)
    h1, self2 = _hop1(p, cidx_flat, features, nodes)
    agg2 = _hop2(h1, npos_flat)
    return _out_proj(agg2, self2, W2_nei, W2_self)


# speedup vs baseline: 1.1645x; 1.0182x over previous
import jax
import jax.numpy as jnp
from jax import lax
from jax.experimental import pallas as pl
from jax.experimental.pallas import tpu as pltpu
from jax.experimental.pallas import tpu_sc as plsc

N_NODES = 50000
D = 256
B = 4096
S = 10
N1 = B * S

NC = 2
NS = 16
NW = NC * NS

L = 16
KD = D // L
KP = D // 32

_MASK_HI = -65536


_ROWS1 = 2000


def _pack_pairs(x):
    a = x[:, :D // 2]
    b = x[:, D // 2:]

    def rnd(v):
        vi = lax.bitcast_convert_type(v, jnp.int32)
        return vi + 0x7FFF + (lax.shift_right_logical(vi, 16) & 1)

    return (rnd(b) & _MASK_HI) | lax.shift_right_logical(rnd(a), 16)


def _transform_body(x_ref, wn_ref, ws_ref, pn_ref, ps_ref):
    x = x_ref[...].astype(jnp.bfloat16)
    dn = (((1,), (1,)), ((), ()))
    pn = lax.dot_general(x, wn_ref[...].astype(jnp.bfloat16), dn,
                         preferred_element_type=jnp.float32)
    ps = lax.dot_general(x, ws_ref[...].astype(jnp.bfloat16), dn,
                         preferred_element_type=jnp.float32)
    pn_ref[...] = _pack_pairs(pn)
    ps_ref[...] = _pack_pairs(ps)


def _transform(features, wn, ws):
    grid = N_NODES // _ROWS1
    return pl.pallas_call(
        _transform_body,
        grid=(grid,),
        in_specs=[
            pl.BlockSpec((_ROWS1, D), lambda i: (i, 0)),
            pl.BlockSpec((D, D), lambda i: (0, 0)),
            pl.BlockSpec((D, D), lambda i: (0, 0)),
        ],
        out_specs=[
            pl.BlockSpec((_ROWS1, D // 2), lambda i: (i, 0)),
            pl.BlockSpec((_ROWS1, D // 2), lambda i: (i, 0)),
        ],
        out_shape=[
            jax.ShapeDtypeStruct((N_NODES, D // 2), jnp.int32),
            jax.ShapeDtypeStruct((N_NODES, D // 2), jnp.int32),
        ],
    )(features, wn, ws)


_C1 = 8
_RPW1 = N1 // NW
_NCH1 = _RPW1 // _C1
_NB = 4
_B_PW = B // NW


def _unpack_lo(x):
    return lax.bitcast_convert_type(lax.shift_left(x, 16), jnp.float32)


def _unpack_hi(x):
    return lax.bitcast_convert_type(x & _MASK_HI, jnp.float32)


def _hop1_body(pn_hbm, ps_hbm, nidx_hbm, sidx_hbm, feat_hbm, nodes_hbm,
               h1_hbm, self2_hbm,
               nidx_v, sidx_v,
               nb0, nb1, nb2, nb3, sb0, sb1, sb2, sb3, ob0, ob1, ob2, ob3,
               s2idx, s2buf,
               g0, g1, g2, g3, o0, o1, o2, o3, s2sem):
    wid = lax.axis_index("s") * NC + lax.axis_index("c")
    base0 = wid * _RPW1
    nbufs = (nb0, nb1, nb2, nb3)
    sbufs = (sb0, sb1, sb2, sb3)
    obufs = (ob0, ob1, ob2, ob3)
    gsems = (g0, g1, g2, g3)
    osems = (o0, o1, o2, o3)

    sbase = wid * _B_PW
    pltpu.sync_copy(nodes_hbm.at[pl.ds(sbase, _B_PW)], s2idx)
    pltpu.async_copy(feat_hbm.at[s2idx], s2buf, s2sem)

    pltpu.sync_copy(nidx_hbm.at[pl.ds(base0 * S, _RPW1 * S)], nidx_v)
    pltpu.sync_copy(sidx_hbm.at[pl.ds(base0, _RPW1)], sidx_v)

    def issue(c, j):
        pltpu.async_copy(pn_hbm.at[nidx_v.at[pl.ds(c * (_C1 * S), _C1 * S)]],
                         nbufs[j], gsems[j])
        pltpu.async_copy(ps_hbm.at[sidx_v.at[pl.ds(c * _C1, _C1)]],
                         sbufs[j], gsems[j])

    def wait_gather(j):
        pltpu.make_async_copy(pn_hbm.at[nidx_v.at[pl.ds(0, _C1 * S)]],
                              nbufs[j], gsems[j]).wait()
        pltpu.make_async_copy(ps_hbm.at[sidx_v.at[pl.ds(0, _C1)]],
                              sbufs[j], gsems[j]).wait()

    for j in range(_NB):
        issue(j, j)

    def step(i, _):
        for j in range(_NB):
            c = _NB * i + j
            wait_gather(j)

            @pl.when(c >= _NB)
            def _w():
                pltpu.make_async_copy(obufs[j], h1_hbm.at[pl.ds(base0, _C1)],
                                      osems[j]).wait()

            nb, sb, ob = nbufs[j], sbufs[j], obufs[j]

            def crow(r, _c):
                for k in range(KP):
                    xs = sb[r, pl.ds(k * L, L)]
                    lo = _unpack_lo(xs)
                    hi = _unpack_hi(xs)
                    for s in range(S):
                        x = nb[r * S + s, pl.ds(k * L, L)]
                        lo = lo + _unpack_lo(x)
                        hi = hi + _unpack_hi(x)
                    ob[r, pl.ds(k * L, L)] = jnp.maximum(lo, 0.0)
                    ob[r, pl.ds(D // 2 + k * L, L)] = jnp.maximum(hi, 0.0)
                return _c

            lax.fori_loop(0, _C1, crow, None)
            pltpu.async_copy(ob, h1_hbm.at[pl.ds(base0 + c * _C1, _C1)],
                             osems[j])

            @pl.when(c + _NB < _NCH1)
            def _i():
                issue(c + _NB, j)
        return _

    lax.fori_loop(0, _NCH1 // _NB, step, None)

    for j in range(_NB):
        pltpu.make_async_copy(obufs[j], h1_hbm.at[pl.ds(base0, _C1)],
                              osems[j]).wait()

    pltpu.make_async_copy(feat_hbm.at[s2idx], s2buf, s2sem).wait()
    pltpu.sync_copy(s2buf, self2_hbm.at[pl.ds(sbase, _B_PW)])


def _hop1(pn, ps, neigh_flat, nodes_l1, features, nodes):
    mesh = plsc.VectorSubcoreMesh(core_axis_name="c", subcore_axis_name="s",
                                  num_cores=NC, num_subcores=NS)
    f = pl.kernel(
        _hop1_body,
        out_type=[
            jax.ShapeDtypeStruct((N1, D), jnp.float32),
            jax.ShapeDtypeStruct((B, D), jnp.float32),
        ],
        mesh=mesh,
        scratch_types=(
            [pltpu.VMEM((_RPW1 * S,), jnp.int32),
             pltpu.VMEM((_RPW1,), jnp.int32)]
            + [pltpu.VMEM((_C1 * S, D // 2), jnp.int32) for _ in range(_NB)]
            + [pltpu.VMEM((_C1, D // 2), jnp.int32) for _ in range(_NB)]
            + [pltpu.VMEM((_C1, D), jnp.float32) for _ in range(_NB)]
            + [pltpu.VMEM((_B_PW,), jnp.int32),
               pltpu.VMEM((_B_PW, D), jnp.float32)]
            + [pltpu.SemaphoreType.DMA] * (2 * _NB + 1)
        ),
    )
    return f(pn, ps, neigh_flat, nodes_l1, features, nodes)


_RPW2 = B // NW
_NCH2 = _RPW2 // _C1


def _hop2_body(h1_hbm, pidx_hbm, agg_hbm,
               pidx_v, nb0, nb1, nb2, nb3, ob0, ob1, ob2, ob3,
               g0, g1, g2, g3, o0, o1, o2, o3):
    wid = lax.axis_index("s") * NC + lax.axis_index("c")
    base0 = wid * _RPW2
    nbufs = (nb0, nb1, nb2, nb3)
    obufs = (ob0, ob1, ob2, ob3)
    gsems = (g0, g1, g2, g3)
    osems = (o0, o1, o2, o3)

    pltpu.sync_copy(pidx_hbm.at[pl.ds(base0 * S, _RPW2 * S)], pidx_v)

    def issue(c, j):
        pltpu.async_copy(h1_hbm.at[pidx_v.at[pl.ds(c * (_C1 * S), _C1 * S)]],
                         nbufs[j], gsems[j])

    def wait_gather(j):
        pltpu.make_async_copy(h1_hbm.at[pidx_v.at[pl.ds(0, _C1 * S)]],
                              nbufs[j], gsems[j]).wait()

    for j in range(_NB):
        issue(j, j)

    def step(i, _):
        for j in range(_NB):
            c = _NB * i + j
            wait_gather(j)

            @pl.when(c >= _NB)
            def _w():
                pltpu.make_async_copy(obufs[j], agg_hbm.at[pl.ds(base0, _C1)],
                                      osems[j]).wait()

            nb, ob = nbufs[j], obufs[j]

            def crow(r, _c):
                for k in range(KD):
                    acc = nb[r * S, pl.ds(k * L, L)]
                    for s in range(1, S):
                        acc = acc + nb[r * S + s, pl.ds(k * L, L)]
                    ob[r, pl.ds(k * L, L)] = acc
                return _c

            lax.fori_loop(0, _C1, crow, None)
            pltpu.async_copy(ob, agg_hbm.at[pl.ds(base0 + c * _C1, _C1)],
                             osems[j])

            @pl.when(c + _NB < _NCH2)
            def _i():
                issue(c + _NB, j)
        return _

    lax.fori_loop(0, _NCH2 // _NB, step, None)

    for j in range(_NB):
        pltpu.make_async_copy(obufs[j], agg_hbm.at[pl.ds(base0, _C1)],
                              osems[j]).wait()


def _hop2(h1p, npos_flat):
    mesh = plsc.VectorSubcoreMesh(core_axis_name="c", subcore_axis_name="s",
                                  num_cores=NC, num_subcores=NS)
    f = pl.kernel(
        _hop2_body,
        out_type=jax.ShapeDtypeStruct((B, D), jnp.float32),
        mesh=mesh,
        scratch_types=(
            [pltpu.VMEM((_RPW2 * S,), jnp.int32)]
            + [pltpu.VMEM((_C1 * S, D), jnp.float32) for _ in range(_NB)]
            + [pltpu.VMEM((_C1, D), jnp.float32) for _ in range(_NB)]
            + [pltpu.SemaphoreType.DMA] * (2 * _NB)
        ),
    )
    return f(h1p, npos_flat)


_ROWS4 = 2048


def _out_body(agg_ref, slf_ref, wn_ref, ws_ref, o_ref):
    dn = (((1,), (1,)), ((), ()))
    x = lax.dot_general(agg_ref[...], wn_ref[...], dn,
                        preferred_element_type=jnp.float32) * (1.0 / S)
    y = lax.dot_general(slf_ref[...], ws_ref[...], dn,
                        preferred_element_type=jnp.float32)
    o_ref[...] = jnp.maximum(x + y, 0.0)


def _out_proj(agg2, self2, w2n, w2s):
    grid = B // _ROWS4
    return pl.pallas_call(
        _out_body,
        grid=(grid,),
        in_specs=[
            pl.BlockSpec((_ROWS4, D), lambda i: (i, 0)),
            pl.BlockSpec((_ROWS4, D), lambda i: (i, 0)),
            pl.BlockSpec((D, D), lambda i: (0, 0)),
            pl.BlockSpec((D, D), lambda i: (0, 0)),
        ],
        out_specs=pl.BlockSpec((_ROWS4, D), lambda i: (i, 0)),
        out_shape=jax.ShapeDtypeStruct((B, D), jnp.float32),
    )(agg2, self2, w2n, w2s)


def kernel(features, nodes, nodes_l1, neigh_l1, neigh_pos, W1_nei, W1_self,
           W2_nei, W2_self):
    nodes = nodes.astype(jnp.int32)
    npos_flat = neigh_pos.reshape(-1)
    neigh_flat = neigh_l1.reshape(-1)

    pn, ps = _transform(features, W1_nei * (1.0 / S), W1_self)
    h1, self2 = _hop1(pn, ps, neigh_flat, nodes_l1, features, nodes)
    agg2 = _hop2(h1, npos_flat)
    return _out_proj(agg2, self2, W2_nei, W2_self)
